# baseline (speedup 1.0000x reference)
import functools

import jax
import jax.numpy as jnp
from jax import lax
from jax.experimental import pallas as pl
from jax.experimental.pallas import tpu as pltpu
from jax.experimental.pallas import tpu_sc as plsc

D = 32
NB = 2
GRP = 4
STEP = 8


@functools.cache
def _make(b, t, v):
    info = plsc.get_sparse_core_info()
    nc, ns = info.num_cores, info.num_subcores
    nw = nc * ns
    assert b % nw == 0
    bat_w = b // nw
    ids_w = bat_w * t
    dpb = t * D // 128
    nfull = t // 16
    tail = t - 16 * nfull
    assert bat_w % STEP == 0 and STEP == 2 * GRP

    mesh = plsc.VectorSubcoreMesh(core_axis_name="c", subcore_axis_name="s")

    @functools.partial(
        pl.kernel,
        mesh=mesh,
        out_type=jax.ShapeDtypeStruct((b * dpb, 128), jnp.float32),
        compiler_params=pltpu.CompilerParams(use_tc_tiling_on_sc=True),
        scratch_types=[
            pltpu.VMEM((ids_w + 16,), jnp.int32),
            pltpu.VMEM((NB, t, D), jnp.float32),
            pltpu.VMEM((2, GRP * dpb, 128), jnp.float32),
            *[pltpu.SemaphoreType.DMA for _ in range(NB + 2)],
        ],
    )
    def gather_kernel(table, idx, out, ids_v, gbuf, dbuf, *sems):
        gsem = sems[:NB]
        osem = sems[NB:]
        wid = lax.axis_index("s") * nc + lax.axis_index("c")

        pltpu.sync_copy(idx.at[pl.ds(wid * ids_w, ids_w)],
                        ids_v.at[pl.ds(0, ids_w)])

        def fire16(i, m, s, n):
            ids = ids_v[pl.ds(t * i + 16 * m, 16)]
            for l in range(n):
                pltpu.async_copy(
                    table.at[pl.ds(ids[l], 1)],
                    gbuf.at[s, pl.ds(16 * m + l, 1)], gsem[s])

        def fire(i, s):
            def body(m, c):
                fire16(i, m, s, 16)
                return c

            lax.fori_loop(0, nfull, body, 0)
            if tail:
                fire16(i, nfull, s, tail)

        fire(0, 0)

        def step(it, carry):
            for u in range(STEP):
                j = it * STEP + u
                s = u % NB
                sn = (u + 1) % NB

                @pl.when(j + 1 < bat_w)
                def _():
                    fire(j + 1, sn)

                pltpu.make_async_copy(
                    table.at[pl.ds(0, t)], gbuf.at[s], gsem[s]).wait()

                d = u // GRP
                if u % GRP == 0:
                    @pl.when(j >= STEP)
                    def _():
                        pltpu.make_async_copy(
                            out.at[pl.ds(0, GRP * dpb)], dbuf.at[d],
                            osem[d]).wait()

                rbase = dpb * (u % GRP)

                def compact(r, c):
                    for q in range(8):
                        val = gbuf[s, 4 * r + q // 2,
                                   pl.ds(16 * (q & 1), 16)]
                        dbuf[d, rbase + r, pl.ds(16 * q, 16)] = val
                    return c

                lax.fori_loop(0, dpb, compact, 0)
                if u % GRP == GRP - 1:
                    first = wid * bat_w + j - (GRP - 1)
                    pltpu.async_copy(
                        dbuf.at[d],
                        out.at[pl.ds(pl.multiple_of(first * dpb, 8),
                                     GRP * dpb)],
                        osem[d])
            return carry

        lax.fori_loop(0, bat_w // STEP, step, 0)
        for d in range(2):
            pltpu.make_async_copy(
                out.at[pl.ds(0, GRP * dpb)], dbuf.at[d], osem[d]).wait()

    return gather_kernel


def kernel(token_ids, time_embeddings_param):
    b, t = token_ids.shape
    idx1 = token_ids.astype(jnp.int32).reshape(-1)
    return _make(b, t, time_embeddings_param.shape[0])(
        time_embeddings_param, idx1).reshape(b, t, D)


# speedup vs baseline: 1.1671x; 1.1671x over previous
import functools

import jax
import jax.numpy as jnp
from jax import lax
from jax.experimental import pallas as pl
from jax.experimental.pallas import tpu as pltpu
from jax.experimental.pallas import tpu_sc as plsc

D = 32
NB = 4
FLY = 3


@functools.cache
def _make(b, t, v):
    info = plsc.get_sparse_core_info()
    nc, ns = info.num_cores, info.num_subcores
    nw = nc * ns
    assert b % nw == 0
    bat_w = b // nw
    ids_w = bat_w * t
    nfull = t // 16
    tail = t - 16 * nfull

    mesh = plsc.VectorSubcoreMesh(core_axis_name="c", subcore_axis_name="s")

    @functools.partial(
        pl.kernel,
        mesh=mesh,
        out_type=jax.ShapeDtypeStruct((b, t, D), jnp.float32),
        compiler_params=pltpu.CompilerParams(use_tc_tiling_on_sc=True),
        scratch_types=[
            pltpu.VMEM((ids_w + 16,), jnp.int32),
            pltpu.VMEM((NB, t, D), jnp.float32),
            *[pltpu.SemaphoreType.DMA for _ in range(2 * NB)],
        ],
    )
    def gather_kernel(table, idx, out, ids_v, obuf, *sems):
        gsem = sems[:NB]
        osem = sems[NB:]
        wid = lax.axis_index("s") * nc + lax.axis_index("c")

        pltpu.sync_copy(idx.at[pl.ds(wid * ids_w, ids_w)],
                        ids_v.at[pl.ds(0, ids_w)])

        def fire16(i, m, s, n):
            ids = ids_v[pl.ds(t * i + 16 * m, 16)]
            for l in range(n):
                pltpu.async_copy(
                    table.at[pl.ds(ids[l], 1)],
                    obuf.at[s, pl.ds(16 * m + l, 1)], gsem[s])

        def fire(i, s):
            def body(m, c):
                fire16(i, m, s, 16)
                return c

            lax.fori_loop(0, nfull, body, 0)
            if tail:
                fire16(i, nfull, s, tail)

        def drain_gather(s):
            pltpu.make_async_copy(
                table.at[pl.ds(0, t)], obuf.at[s], gsem[s]).wait()

        for s in range(FLY):
            fire(s, s)

        def step(i, carry):
            for u in range(NB):
                j = i * NB + u
                s = u % NB
                sn = (u + FLY) % NB
                nj = j + FLY

                @pl.when(nj < bat_w)
                def _():
                    @pl.when(nj >= NB)
                    def _():
                        pltpu.make_async_copy(
                            obuf.at[sn], out.at[wid * bat_w], osem[sn]).wait()
                    fire(nj, sn)

                drain_gather(s)
                pltpu.async_copy(obuf.at[s], out.at[wid * bat_w + j], osem[s])
            return carry

        lax.fori_loop(0, bat_w // NB, step, 0)
        for s in range(NB):
            pltpu.make_async_copy(
                obuf.at[s], out.at[wid * bat_w], osem[s]).wait()

    return gather_kernel


def kernel(token_ids, time_embeddings_param):
    b, t = token_ids.shape
    idx1 = token_ids.astype(jnp.int32).reshape(-1)
    return _make(b, t, time_embeddings_param.shape[0])(
        time_embeddings_param, idx1)
